# baseline (speedup 1.0000x reference)
import functools

import jax
import jax.numpy as jnp
from jax import lax
from jax.experimental import pallas as pl
from jax.experimental.pallas import tpu as pltpu
from jax.experimental.pallas import tpu_sc as plsc

N, D = 8192, 768
H1, H2, E = 2048, 2048, 256
K = 8192
BETA = 0.25

BN = 512
NB = N // BN
BNE = 1024
NBE = N // BNE

BF = jnp.bfloat16
F32 = jnp.float32


def _mm_t(a, b):
    return lax.dot_general(a.astype(BF), b.astype(BF), (((1,), (1,)), ((), ())),
                           preferred_element_type=F32)


def _enc_body(x_ref, We1_ref, be1_ref, We2_ref, be2_ref, We3_ref, be3_ref,
              z_ref):
    h = jnp.maximum(_mm_t(x_ref[...], We1_ref[...]) + be1_ref[...], 0.0)
    h = jnp.maximum(_mm_t(h, We2_ref[...]) + be2_ref[...], 0.0)
    z_ref[...] = _mm_t(h, We3_ref[...]) + be3_ref[...]


def _encoder(x, We1, be1, We2, be2, We3, be3):
    full = lambda s: pl.BlockSpec(s, lambda i: (0,) * len(s))
    return pl.pallas_call(
        _enc_body,
        grid=(NBE,),
        in_specs=[
            pl.BlockSpec((BNE, D), lambda i: (i, 0)),
            full((H1, D)), full((1, H1)),
            full((H2, H1)), full((1, H2)),
            full((E, H2)), full((1, E)),
        ],
        out_specs=pl.BlockSpec((BNE, E), lambda i: (i, 0)),
        out_shape=jax.ShapeDtypeStruct((N, E), F32),
    )(x.astype(BF), We1.astype(BF), be1.reshape(1, H1), We2.astype(BF),
      be2.reshape(1, H2), We3.astype(BF), be3.reshape(1, E))


def _enorm_body(emb_ref, en_ref):
    emb = emb_ref[...]
    en_ref[...] = lax.dot_general(
        jnp.ones((1, E), F32), emb * emb, (((1,), (1,)), ((), ())),
        preferred_element_type=F32, precision=lax.Precision.HIGHEST)


def _enorm(embedding):
    return pl.pallas_call(
        _enorm_body,
        grid=(1,),
        in_specs=[pl.BlockSpec((K, E), lambda i: (0, 0))],
        out_specs=pl.BlockSpec((1, K), lambda i: (0, 0)),
        out_shape=jax.ShapeDtypeStruct((1, K), F32),
    )(embedding)


def _vq_body(z_ref, embbf_ref, embf_ref, idx_ref, gidx_ref, fpos_ref,
             dsum_ref, en_ref):
    @pl.when(pl.program_id(0) == 0)
    def _():
        emb = embf_ref[...]
        en_ref[...] = lax.dot_general(
            jnp.ones((1, E), F32), emb * emb, (((1,), (1,)), ((), ())),
            preferred_element_type=F32, precision=lax.Precision.HIGHEST)

    z = z_ref[...]
    embbf = embbf_ref[...]
    scores = lax.dot_general(z.astype(BF), embbf, (((1,), (1,)), ((), ())),
                             preferred_element_type=F32)
    znorm = jnp.sum(z * z, axis=1, keepdims=True)
    d = znorm - 2.0 * scores + en_ref[...]
    dmin = jnp.min(d, axis=1, keepdims=True)
    iota = lax.broadcasted_iota(jnp.int32, d.shape, 1)
    idx = jnp.min(jnp.where(d <= dmin, iota, K), axis=1)
    idx_ref[0, 0, :] = idx
    riot = lax.broadcasted_iota(jnp.int32, (BN, BN), 1)
    eqm = idx[:, None] == idx[None, :]
    fpos = jnp.min(jnp.where(eqm, riot, BN), axis=1)
    rows = lax.broadcasted_iota(jnp.int32, (BN,), 0)
    grow = pl.program_id(0) * BN + rows
    gidx = jnp.where(fpos == rows, idx, grow)
    gidx_ref[0, 0, :] = gidx
    fpos_ref[0, 0, :] = fpos
    dsum_ref[0, 0, :] = jnp.broadcast_to(jnp.sum(dmin), (128,))


def _vq(z_e, emb_bf, embedding):
    n = z_e.shape[0]
    nb = n // BN
    idx3, gidx3, fpos3, dsum = pl.pallas_call(
        _vq_body,
        grid=(nb,),
        in_specs=[
            pl.BlockSpec((BN, E), lambda i: (i, 0)),
            pl.BlockSpec((K, E), lambda i: (0, 0)),
            pl.BlockSpec((K, E), lambda i: (0, 0)),
        ],
        out_specs=[
            pl.BlockSpec((1, 1, BN), lambda i: (i, 0, 0)),
            pl.BlockSpec((1, 1, BN), lambda i: (i, 0, 0)),
            pl.BlockSpec((1, 1, BN), lambda i: (i, 0, 0)),
            pl.BlockSpec((1, 1, 128), lambda i: (i, 0, 0)),
        ],
        out_shape=[
            jax.ShapeDtypeStruct((nb, 1, BN), jnp.int32),
            jax.ShapeDtypeStruct((nb, 1, BN), jnp.int32),
            jax.ShapeDtypeStruct((nb, 1, BN), jnp.int32),
            jax.ShapeDtypeStruct((nb, 1, 128), F32),
        ],
        scratch_shapes=[pltpu.VMEM((1, K), F32)],
    )(z_e, emb_bf, embedding)
    return idx3.reshape(n), gidx3.reshape(n), fpos3, dsum


def _sc_gather(table, idx):
    info = plsc.get_sparse_core_info()
    n = idx.shape[0]
    nw = info.num_cores * info.num_subcores
    b_per_w = n // nw
    c = 128
    n_chunks = b_per_w // c
    mesh = plsc.VectorSubcoreMesh(core_axis_name="c", subcore_axis_name="s")

    @functools.partial(
        pl.kernel, mesh=mesh,
        out_type=jax.ShapeDtypeStruct((n, E), F32),
        scratch_types=[
            pltpu.VMEM((n_chunks, c), jnp.int32),
            pltpu.VMEM((b_per_w, E), F32),
            pltpu.SemaphoreType.DMA,
        ],
    )
    def k(table_hbm, idx_hbm, out_hbm, idx_v, rows_v, sem):
        wid = lax.axis_index("s") * info.num_cores + lax.axis_index("c")
        base = wid * b_per_w
        pltpu.sync_copy(idx_hbm.at[wid], idx_v)
        copies = [
            pltpu.async_copy(table_hbm.at[idx_v.at[j]],
                             rows_v.at[pl.ds(j * c, c)], sem)
            for j in range(n_chunks)
        ]
        for cp in copies:
            cp.wait()
        pltpu.sync_copy(rows_v, out_hbm.at[pl.ds(base, b_per_w)])

    return k(table, idx.reshape(nw, n_chunks, c))


def _dec_body(q_ref, fpos_ref, x_ref, Wd1_ref, bd1_ref, Wd2_ref, bd2_ref,
              Wd3_ref, bd3_ref, recon_ref, rsum_ref):
    fpos = fpos_ref[0, 0, :]
    lio = lax.broadcasted_iota(jnp.int32, (BN, BN), 1)
    lonehot = (fpos[:, None] == lio).astype(BF)
    qbf = lax.dot_general(lonehot, q_ref[...].astype(BF),
                          (((1,), (0,)), ((), ())),
                          preferred_element_type=F32).astype(BF)
    h = jnp.maximum(_mm_t(qbf, Wd1_ref[...]) + bd1_ref[...], 0.0)
    h = jnp.maximum(_mm_t(h, Wd2_ref[...]) + bd2_ref[...], 0.0)
    recon = _mm_t(h, Wd3_ref[...]) + bd3_ref[...]
    recon_ref[...] = recon
    dr = recon - x_ref[...]
    rsum_ref[0, 0, :] = jnp.broadcast_to(jnp.sum(dr * dr), (128,))


def _decoder(q_gathered, fpos3, x, Wd1, bd1, Wd2, bd2, Wd3, bd3):
    full = lambda s: pl.BlockSpec(s, lambda i: (0,) * len(s))
    return pl.pallas_call(
        _dec_body,
        grid=(NB,),
        in_specs=[
            pl.BlockSpec((BN, E), lambda i: (i, 0)),
            pl.BlockSpec((1, 1, BN), lambda i: (i, 0, 0)),
            pl.BlockSpec((BN, D), lambda i: (i, 0)),
            full((H2, E)), full((1, H2)),
            full((H1, H2)), full((1, H1)),
            full((D, H1)), full((1, D)),
        ],
        out_specs=[
            pl.BlockSpec((BN, D), lambda i: (i, 0)),
            pl.BlockSpec((1, 1, 128), lambda i: (i, 0, 0)),
        ],
        out_shape=[
            jax.ShapeDtypeStruct((N, D), F32),
            jax.ShapeDtypeStruct((NB, 1, 128), F32),
        ],
    )(q_gathered, fpos3, x, Wd1.astype(BF), bd1.reshape(1, H2),
      Wd2.astype(BF), bd2.reshape(1, H1), Wd3.astype(BF), bd3.reshape(1, D))


def kernel(x, We1, be1, We2, be2, We3, be3, Wd1, bd1, Wd2, bd2, Wd3, bd3,
           embedding):
    z_e = _encoder(x, We1, be1, We2, be2, We3, be3)
    emb_bf = embedding.astype(BF)
    hn = N // 2
    parts = [
        _vq(lax.slice_in_dim(z_e, h * hn, (h + 1) * hn), emb_bf, embedding)
        for h in range(2)
    ]
    q_halves = [_sc_gather(embedding, p[1]) for p in parts]
    indices = jnp.concatenate([p[0] for p in parts])
    fpos3 = jnp.concatenate([p[2] for p in parts])
    dsum = jnp.concatenate([p[3] for p in parts])
    q_gathered = jnp.concatenate(q_halves)
    recon, rsum = _decoder(q_gathered, fpos3, x, Wd1, bd1, Wd2, bd2, Wd3, bd3)
    recon_loss = jnp.sum(rsum[:, 0, 0]) / (N * D)
    commitment_loss = BETA * (jnp.sum(dsum[:, 0, 0]) / (N * E))
    loss = recon_loss + commitment_loss
    return loss, recon_loss, indices, recon


# speedup vs baseline: 1.0771x; 1.0771x over previous
import functools

import jax
import jax.numpy as jnp
from jax import lax
from jax.experimental import pallas as pl
from jax.experimental.pallas import tpu as pltpu
from jax.experimental.pallas import tpu_sc as plsc

N, D = 8192, 768
H1, H2, E = 2048, 2048, 256
K = 8192
BETA = 0.25

BN = 512
NB = N // BN
BNE = 1024
NBE = N // BNE

BF = jnp.bfloat16
F32 = jnp.float32


def _mm_t(a, b):
    return lax.dot_general(a.astype(BF), b.astype(BF), (((1,), (1,)), ((), ())),
                           preferred_element_type=F32)


def _enc_body(x_ref, We1_ref, be1_ref, We2_ref, be2_ref, We3_ref, be3_ref,
              z_ref):
    h = jnp.maximum(_mm_t(x_ref[...], We1_ref[...]) + be1_ref[...], 0.0)
    h = jnp.maximum(_mm_t(h, We2_ref[...]) + be2_ref[...], 0.0)
    z_ref[...] = _mm_t(h, We3_ref[...]) + be3_ref[...]


def _encoder(x, We1, be1, We2, be2, We3, be3):
    full = lambda s: pl.BlockSpec(s, lambda i: (0,) * len(s))
    return pl.pallas_call(
        _enc_body,
        grid=(NBE,),
        in_specs=[
            pl.BlockSpec((BNE, D), lambda i: (i, 0)),
            full((H1, D)), full((1, H1)),
            full((H2, H1)), full((1, H2)),
            full((E, H2)), full((1, E)),
        ],
        out_specs=pl.BlockSpec((BNE, E), lambda i: (i, 0)),
        out_shape=jax.ShapeDtypeStruct((N, E), F32),
    )(x.astype(BF), We1.astype(BF), be1.reshape(1, H1), We2.astype(BF),
      be2.reshape(1, H2), We3.astype(BF), be3.reshape(1, E))


def _vq_body(z_ref, embbf_ref, embf_ref, idx_ref, gidx_ref, fpos_ref,
             dsum_ref, en_ref):
    @pl.when(pl.program_id(0) == 0)
    def _():
        emb = embf_ref[...]
        en_ref[...] = lax.dot_general(
            jnp.ones((1, E), F32), emb * emb, (((1,), (1,)), ((), ())),
            preferred_element_type=F32, precision=lax.Precision.HIGHEST)

    z = z_ref[...]
    embbf = embbf_ref[...]
    scores = lax.dot_general(z.astype(BF), embbf, (((1,), (1,)), ((), ())),
                             preferred_element_type=F32)
    znorm = jnp.sum(z * z, axis=1, keepdims=True)
    d = znorm - 2.0 * scores + en_ref[...]
    dmin = jnp.min(d, axis=1, keepdims=True)
    iota = lax.broadcasted_iota(jnp.int32, d.shape, 1)
    idx = jnp.min(jnp.where(d <= dmin, iota, K), axis=1)
    idx_ref[0, 0, :] = idx
    riot = lax.broadcasted_iota(jnp.int32, (BN, BN), 1)
    eqm = idx[:, None] == idx[None, :]
    fpos = jnp.min(jnp.where(eqm, riot, BN), axis=1)
    rows = lax.broadcasted_iota(jnp.int32, (BN,), 0)
    grow = pl.program_id(0) * BN + rows
    gidx = jnp.where(fpos == rows, idx, grow)
    gidx_ref[0, 0, :] = gidx
    fpos_ref[0, 0, :] = fpos
    dsum_ref[0, 0, :] = jnp.broadcast_to(jnp.sum(dmin), (128,))


def _vq(z_e, emb_bf, embedding):
    n = z_e.shape[0]
    nb = n // BN
    idx3, gidx3, fpos3, dsum = pl.pallas_call(
        _vq_body,
        grid=(nb,),
        in_specs=[
            pl.BlockSpec((BN, E), lambda i: (i, 0)),
            pl.BlockSpec((K, E), lambda i: (0, 0)),
            pl.BlockSpec((K, E), lambda i: (0, 0)),
        ],
        out_specs=[
            pl.BlockSpec((1, 1, BN), lambda i: (i, 0, 0)),
            pl.BlockSpec((1, 1, BN), lambda i: (i, 0, 0)),
            pl.BlockSpec((1, 1, BN), lambda i: (i, 0, 0)),
            pl.BlockSpec((1, 1, 128), lambda i: (i, 0, 0)),
        ],
        out_shape=[
            jax.ShapeDtypeStruct((nb, 1, BN), jnp.int32),
            jax.ShapeDtypeStruct((nb, 1, BN), jnp.int32),
            jax.ShapeDtypeStruct((nb, 1, BN), jnp.int32),
            jax.ShapeDtypeStruct((nb, 1, 128), F32),
        ],
        scratch_shapes=[pltpu.VMEM((1, K), F32)],
    )(z_e, emb_bf, embedding)
    return idx3.reshape(n), gidx3.reshape(n), fpos3, dsum


def _sc_gather(table, idx):
    info = plsc.get_sparse_core_info()
    n = idx.shape[0]
    nw = info.num_cores * info.num_subcores
    b_per_w = n // nw
    c = 128
    n_chunks = b_per_w // c
    mesh = plsc.VectorSubcoreMesh(core_axis_name="c", subcore_axis_name="s")

    @functools.partial(
        pl.kernel, mesh=mesh,
        out_type=jax.ShapeDtypeStruct((n, E), F32),
        scratch_types=[
            pltpu.VMEM((n_chunks, c), jnp.int32),
            pltpu.VMEM((b_per_w, E), F32),
            pltpu.SemaphoreType.DMA,
        ],
    )
    def k(table_hbm, idx_hbm, out_hbm, idx_v, rows_v, sem):
        wid = lax.axis_index("s") * info.num_cores + lax.axis_index("c")
        base = wid * b_per_w
        pltpu.sync_copy(idx_hbm.at[wid], idx_v)
        copies = [
            pltpu.async_copy(table_hbm.at[idx_v.at[j]],
                             rows_v.at[pl.ds(j * c, c)], sem)
            for j in range(n_chunks)
        ]
        for cp in copies:
            cp.wait()
        pltpu.sync_copy(rows_v, out_hbm.at[pl.ds(base, b_per_w)])

    return k(table, idx.reshape(nw, n_chunks, c))


def _dec_body(q_ref, fpos_ref, x_ref, Wd1_ref, bd1_ref, Wd2_ref, bd2_ref,
              Wd3_ref, bd3_ref, recon_ref, rsum_ref):
    fpos = fpos_ref[0, 0, :]
    lio = lax.broadcasted_iota(jnp.int32, (BN, BN), 1)
    lonehot = (fpos[:, None] == lio).astype(BF)
    qbf = lax.dot_general(lonehot, q_ref[...].astype(BF),
                          (((1,), (0,)), ((), ())),
                          preferred_element_type=F32).astype(BF)
    h = jnp.maximum(_mm_t(qbf, Wd1_ref[...]) + bd1_ref[...], 0.0)
    h = jnp.maximum(_mm_t(h, Wd2_ref[...]) + bd2_ref[...], 0.0)
    recon = _mm_t(h, Wd3_ref[...]) + bd3_ref[...]
    recon_ref[...] = recon
    dr = recon - x_ref[...]
    rsum_ref[0, 0, :] = jnp.broadcast_to(jnp.sum(dr * dr), (128,))


def _decoder(q_gathered, fpos3, x, Wd1, bd1, Wd2, bd2, Wd3, bd3):
    full = lambda s: pl.BlockSpec(s, lambda i: (0,) * len(s))
    return pl.pallas_call(
        _dec_body,
        grid=(NB,),
        in_specs=[
            pl.BlockSpec((BN, E), lambda i: (i, 0)),
            pl.BlockSpec((1, 1, BN), lambda i: (i, 0, 0)),
            pl.BlockSpec((BN, D), lambda i: (i, 0)),
            full((H2, E)), full((1, H2)),
            full((H1, H2)), full((1, H1)),
            full((D, H1)), full((1, D)),
        ],
        out_specs=[
            pl.BlockSpec((BN, D), lambda i: (i, 0)),
            pl.BlockSpec((1, 1, 128), lambda i: (i, 0, 0)),
        ],
        out_shape=[
            jax.ShapeDtypeStruct((N, D), F32),
            jax.ShapeDtypeStruct((NB, 1, 128), F32),
        ],
    )(q_gathered, fpos3, x, Wd1.astype(BF), bd1.reshape(1, H2),
      Wd2.astype(BF), bd2.reshape(1, H1), Wd3.astype(BF), bd3.reshape(1, D))


def kernel(x, We1, be1, We2, be2, We3, be3, Wd1, bd1, Wd2, bd2, Wd3, bd3,
           embedding):
    z_e = _encoder(x, We1, be1, We2, be2, We3, be3)
    indices, gidx, fpos3, dsum = _vq(z_e, embedding.astype(BF), embedding)
    q_gathered = _sc_gather(embedding, gidx)
    recon, rsum = _decoder(q_gathered, fpos3, x, Wd1, bd1, Wd2, bd2, Wd3, bd3)
    recon_loss = jnp.sum(rsum[:, 0, 0]) / (N * D)
    commitment_loss = BETA * (jnp.sum(dsum[:, 0, 0]) / (N * E))
    loss = recon_loss + commitment_loss
    return loss, recon_loss, indices, recon
